# Initial kernel scaffold; baseline (speedup 1.0000x reference)
import functools

import jax
import jax.numpy as jnp
from jax import lax
from jax.experimental import pallas as pl
from jax.experimental.pallas import tpu as pltpu
from jax.experimental.pallas import tpu_sc as plsc

N_NODES = 10000
N_EDGES = 320000
D_FEAT = 128
D_HID = 32

NC = 2
NS = 16
NW = NC * NS
L = 16


_E_PER_TILE = N_EDGES // NW
_G_DEG = _E_PER_TILE // L
_ZN = N_NODES // L


def _sc_degree_body(dst_hbm, ew_hbm, degp_hbm, dst_v, ew_v, deg_v):
    wid = lax.axis_index("s") * NC + lax.axis_index("c")
    base = wid * _E_PER_TILE
    pltpu.sync_copy(dst_hbm.at[pl.ds(base, _E_PER_TILE)], dst_v)
    pltpu.sync_copy(ew_hbm.at[pl.ds(base, _E_PER_TILE)], ew_v)

    def zero(i, _):
        deg_v[pl.ds(i * L, L)] = jnp.zeros((L,), jnp.float32)
        return 0

    lax.fori_loop(0, _ZN, zero, 0)

    def group(g, _):
        s = pl.ds(g * L, L)
        plsc.addupdate_scatter(deg_v, [dst_v[s]], ew_v[s])
        return 0

    lax.fori_loop(0, _G_DEG, group, 0)
    pltpu.sync_copy(deg_v, degp_hbm.at[wid])


def _sc_degree(dst, ew):
    mesh = plsc.VectorSubcoreMesh(core_axis_name="c", subcore_axis_name="s")
    return pl.kernel(
        _sc_degree_body,
        mesh=mesh,
        out_type=jax.ShapeDtypeStruct((NW, N_NODES), jnp.float32),
        scratch_types=[
            pltpu.VMEM((_E_PER_TILE,), jnp.int32),
            pltpu.VMEM((_E_PER_TILE,), jnp.float32),
            pltpu.VMEM((N_NODES,), jnp.float32),
        ],
    )(dst, ew)


def _tc_prep_body(x_ref, w1_ref, degp_ref, xwt_ref, dinv_ref):
    deg = jnp.sum(degp_ref[...], axis=0, keepdims=True) + 1.0
    dinv = jnp.where(deg > 0, lax.rsqrt(jnp.maximum(deg, 1e-12)), 0.0)
    xwt = lax.dot_general(
        w1_ref[...], x_ref[...], (((0,), (1,)), ((), ())),
        preferred_element_type=jnp.float32,
    )
    xwt_ref[...] = xwt * dinv
    dinv_ref[...] = dinv


def _tc_prep(x, w1, degp):
    return pl.pallas_call(
        _tc_prep_body,
        out_shape=[
            jax.ShapeDtypeStruct((D_HID, N_NODES), jnp.float32),
            jax.ShapeDtypeStruct((1, N_NODES), jnp.float32),
        ],
    )(x, w1, degp)


FC = 4
N_FC = D_HID // FC
N_ES = NW // N_FC
_E_PER_SLAB = N_EDGES // N_ES
_CH = 8000
_N_CH = _E_PER_SLAB // _CH
_G_MSG = _CH // L
_CHUNK_W = FC * N_NODES
_ZA = _CHUNK_W // L


def _sc_msg_body(xwt_hbm, dinv_hbm, src_hbm, dst_hbm, ew_hbm, accp_hbm,
                 xwt_v, dinv_v, acc_v, src_v, dst_v, ew_v):
    wid = lax.axis_index("s") * NC + lax.axis_index("c")
    fc = wid % N_FC
    es = wid // N_FC

    pltpu.sync_copy(xwt_hbm.at[pl.ds(fc * _CHUNK_W, _CHUNK_W)], xwt_v)
    pltpu.sync_copy(dinv_hbm, dinv_v)

    def zero(i, _):
        acc_v[pl.ds(i * L, L)] = jnp.zeros((L,), jnp.float32)
        return 0

    lax.fori_loop(0, _ZA, zero, 0)

    ebase = es * _E_PER_SLAB

    def chunk(k, _):
        cb = ebase + k * _CH
        pltpu.sync_copy(src_hbm.at[pl.ds(cb, _CH)], src_v)
        pltpu.sync_copy(dst_hbm.at[pl.ds(cb, _CH)], dst_v)
        pltpu.sync_copy(ew_hbm.at[pl.ds(cb, _CH)], ew_v)

        def group(g, _):
            s = pl.ds(g * L, L)
            vs = src_v[s]
            vd = dst_v[s]
            scale = ew_v[s] * plsc.load_gather(dinv_v, [vd])
            for c in range(FC):
                off = jnp.int32(c * N_NODES)
                vals = plsc.load_gather(xwt_v, [vs + off])
                plsc.addupdate_scatter(acc_v, [vd + off], vals * scale)
            return 0

        lax.fori_loop(0, _G_MSG, group, 0)
        return 0

    lax.fori_loop(0, _N_CH, chunk, 0)
    pltpu.sync_copy(acc_v, accp_hbm.at[es, fc])


def _sc_msg(xwt_flat, dinv_flat, src, dst, ew):
    mesh = plsc.VectorSubcoreMesh(core_axis_name="c", subcore_axis_name="s")
    return pl.kernel(
        _sc_msg_body,
        mesh=mesh,
        out_type=jax.ShapeDtypeStruct((N_ES, N_FC, _CHUNK_W), jnp.float32),
        scratch_types=[
            pltpu.VMEM((_CHUNK_W,), jnp.float32),
            pltpu.VMEM((N_NODES,), jnp.float32),
            pltpu.VMEM((_CHUNK_W,), jnp.float32),
            pltpu.VMEM((_CH,), jnp.int32),
            pltpu.VMEM((_CH,), jnp.int32),
            pltpu.VMEM((_CH,), jnp.float32),
        ],
    )(xwt_flat, dinv_flat, src, dst, ew)


def _tc_out_body(accp_ref, xwt_ref, dinv_ref, b1_ref, w2_ref, b2_ref, y_ref):
    h = jnp.sum(accp_ref[...], axis=0)
    h = h + dinv_ref[...] * xwt_ref[...] + b1_ref[...]
    h = jnp.maximum(h, 0.0)
    y = lax.dot_general(
        w2_ref[...], h, (((0,), (0,)), ((), ())),
        preferred_element_type=jnp.float32,
    )
    y_ref[...] = y + b2_ref[...]


def _tc_out(accp, xwt, dinv, b1, w2, b2):
    return pl.pallas_call(
        _tc_out_body,
        out_shape=jax.ShapeDtypeStruct((1, N_NODES), jnp.float32),
    )(accp, xwt, dinv, b1, w2, b2)


def kernel(x, edge_index, edge_weight, W1, b1, W2, b2):
    src = edge_index[0].astype(jnp.int32)
    dst = edge_index[1].astype(jnp.int32)
    ew = edge_weight.astype(jnp.float32)

    degp = _sc_degree(dst, ew)
    xwt, dinv = _tc_prep(x, W1, degp)
    accp = _sc_msg(xwt.reshape(-1), dinv.reshape(-1), src, dst, ew)
    accp = accp.reshape(N_ES, D_HID, N_NODES)
    y = _tc_out(accp, xwt, dinv, b1.reshape(D_HID, 1), W2, b2.reshape(1, 1))
    return y.reshape(N_NODES, 1)


# speedup vs baseline: 29.5192x; 29.5192x over previous
import functools

import jax
import jax.numpy as jnp
from jax import lax
from jax.experimental import pallas as pl
from jax.experimental.pallas import tpu as pltpu
from jax.experimental.pallas import tpu_sc as plsc

N_NODES = 10000
N_EDGES = 320000
D_FEAT = 128
D_HID = 32

NC = 2
NS = 16
NW = NC * NS
L = 16


_E_PER_TILE = N_EDGES // NW
_G_DEG = _E_PER_TILE // L
_ZN = N_NODES // L


def _sc_degree_body(dst_hbm, ew_hbm, degp_hbm, dst_v, ew_v, deg_v):
    wid = lax.axis_index("s") * NC + lax.axis_index("c")
    base = wid * _E_PER_TILE
    pltpu.sync_copy(dst_hbm.at[pl.ds(base, _E_PER_TILE)], dst_v)
    pltpu.sync_copy(ew_hbm.at[pl.ds(base, _E_PER_TILE)], ew_v)

    def zero(i, _):
        deg_v[pl.ds(i * L, L)] = jnp.zeros((L,), jnp.float32)
        return 0

    lax.fori_loop(0, _ZN, zero, 0)

    def group(g, _):
        s = pl.ds(g * L, L)
        plsc.addupdate_scatter(deg_v, [dst_v[s]], ew_v[s])
        return 0

    lax.fori_loop(0, _G_DEG, group, 0)
    pltpu.sync_copy(deg_v, degp_hbm.at[wid])


def _sc_degree(dst, ew):
    mesh = plsc.VectorSubcoreMesh(core_axis_name="c", subcore_axis_name="s")
    return pl.kernel(
        _sc_degree_body,
        mesh=mesh,
        compiler_params=pltpu.CompilerParams(needs_layout_passes=False),
        out_type=jax.ShapeDtypeStruct((NW, N_NODES), jnp.float32),
        scratch_types=[
            pltpu.VMEM((_E_PER_TILE,), jnp.int32),
            pltpu.VMEM((_E_PER_TILE,), jnp.float32),
            pltpu.VMEM((N_NODES,), jnp.float32),
        ],
    )(dst, ew)


def _tc_prep_body(x_ref, w1_ref, degp_ref, xwt_ref, dinv_ref):
    deg = jnp.sum(degp_ref[...], axis=0, keepdims=True) + 1.0
    dinv = jnp.where(deg > 0, lax.rsqrt(jnp.maximum(deg, 1e-12)), 0.0)
    xwt = lax.dot_general(
        w1_ref[...], x_ref[...], (((0,), (1,)), ((), ())),
        preferred_element_type=jnp.float32,
    )
    xwt_ref[...] = xwt * dinv
    dinv_ref[...] = dinv


def _tc_prep(x, w1, degp):
    return pl.pallas_call(
        _tc_prep_body,
        out_shape=[
            jax.ShapeDtypeStruct((D_HID, N_NODES), jnp.float32),
            jax.ShapeDtypeStruct((1, N_NODES), jnp.float32),
        ],
    )(x, w1, degp)


FC = 4
N_FC = D_HID // FC
N_ES = NW // N_FC
_E_PER_SLAB = N_EDGES // N_ES
_CH = 8000
_N_CH = _E_PER_SLAB // _CH
_G_MSG = _CH // L
_CHUNK_W = FC * N_NODES
_ZA = _CHUNK_W // L


def _sc_msg_body(xwt_hbm, dinv_hbm, src_hbm, dst_hbm, ew_hbm, accp_hbm,
                 xwt_v, dinv_v, acc_v, src_v, dst_v, ew_v):
    wid = lax.axis_index("s") * NC + lax.axis_index("c")
    fc = wid % N_FC
    es = wid // N_FC

    pltpu.sync_copy(xwt_hbm.at[pl.ds(fc * _CHUNK_W, _CHUNK_W)], xwt_v)
    pltpu.sync_copy(dinv_hbm, dinv_v)

    def zero(i, _):
        acc_v[pl.ds(i * L, L)] = jnp.zeros((L,), jnp.float32)
        return 0

    lax.fori_loop(0, _ZA, zero, 0)

    ebase = es * _E_PER_SLAB

    def chunk(k, _):
        cb = ebase + k * _CH
        pltpu.sync_copy(src_hbm.at[pl.ds(cb, _CH)], src_v)
        pltpu.sync_copy(dst_hbm.at[pl.ds(cb, _CH)], dst_v)
        pltpu.sync_copy(ew_hbm.at[pl.ds(cb, _CH)], ew_v)

        def group(g, _):
            s = pl.ds(g * L, L)
            vs = src_v[s]
            vd = dst_v[s]
            scale = ew_v[s] * plsc.load_gather(dinv_v, [vd])
            for c in range(FC):
                off = jnp.int32(c * N_NODES)
                vals = plsc.load_gather(xwt_v, [vs + off])
                plsc.addupdate_scatter(acc_v, [vd + off], vals * scale)
            return 0

        lax.fori_loop(0, _G_MSG, group, 0)
        return 0

    lax.fori_loop(0, _N_CH, chunk, 0)
    pltpu.sync_copy(acc_v, accp_hbm.at[es, fc])


def _sc_msg(xwt_flat, dinv_flat, src, dst, ew):
    mesh = plsc.VectorSubcoreMesh(core_axis_name="c", subcore_axis_name="s")
    return pl.kernel(
        _sc_msg_body,
        mesh=mesh,
        compiler_params=pltpu.CompilerParams(needs_layout_passes=False),
        out_type=jax.ShapeDtypeStruct((N_ES, N_FC, _CHUNK_W), jnp.float32),
        scratch_types=[
            pltpu.VMEM((_CHUNK_W,), jnp.float32),
            pltpu.VMEM((N_NODES,), jnp.float32),
            pltpu.VMEM((_CHUNK_W,), jnp.float32),
            pltpu.VMEM((_CH,), jnp.int32),
            pltpu.VMEM((_CH,), jnp.int32),
            pltpu.VMEM((_CH,), jnp.float32),
        ],
    )(xwt_flat, dinv_flat, src, dst, ew)


def _tc_out_body(accp_ref, xwt_ref, dinv_ref, b1_ref, w2_ref, b2_ref, y_ref):
    h = jnp.sum(accp_ref[...], axis=0)
    h = h + dinv_ref[...] * xwt_ref[...] + b1_ref[...]
    h = jnp.maximum(h, 0.0)
    y = lax.dot_general(
        w2_ref[...], h, (((0,), (0,)), ((), ())),
        preferred_element_type=jnp.float32,
    )
    y_ref[...] = y + b2_ref[...]


def _tc_out(accp, xwt, dinv, b1, w2, b2):
    return pl.pallas_call(
        _tc_out_body,
        out_shape=jax.ShapeDtypeStruct((1, N_NODES), jnp.float32),
    )(accp, xwt, dinv, b1, w2, b2)


def kernel(x, edge_index, edge_weight, W1, b1, W2, b2):
    src = edge_index[0].astype(jnp.int32)
    dst = edge_index[1].astype(jnp.int32)
    ew = edge_weight.astype(jnp.float32)

    degp = _sc_degree(dst, ew)
    xwt, dinv = _tc_prep(x, W1, degp)
    accp = _sc_msg(xwt.reshape(-1), dinv.reshape(-1), src, dst, ew)
    accp = accp.reshape(N_ES, D_HID, N_NODES)
    y = _tc_out(accp, xwt, dinv, b1.reshape(D_HID, 1), W2, b2.reshape(1, 1))
    return y.reshape(N_NODES, 1)


# speedup vs baseline: 52.3133x; 1.7722x over previous
import functools

import jax
import jax.numpy as jnp
from jax import lax
from jax.experimental import pallas as pl
from jax.experimental.pallas import tpu as pltpu
from jax.experimental.pallas import tpu_sc as plsc

N_NODES = 10000
N_EDGES = 320000
D_FEAT = 128
D_HID = 32

NC = 2
NS = 16
NW = NC * NS
L = 16


_E_PER_TILE = N_EDGES // NW
_G_DEG = _E_PER_TILE // L
_ZN = N_NODES // L


def _sc_degree_body(dst_hbm, ew_hbm, degp_hbm, dst_v, ew_v, deg_v):
    wid = lax.axis_index("s") * NC + lax.axis_index("c")
    base = wid * _E_PER_TILE
    pltpu.sync_copy(dst_hbm.at[pl.ds(base, _E_PER_TILE)], dst_v)
    pltpu.sync_copy(ew_hbm.at[pl.ds(base, _E_PER_TILE)], ew_v)

    @plsc.parallel_loop(0, _ZN, unroll=8)
    def _zero(i):
        deg_v[pl.ds(i * L, L)] = jnp.zeros((L,), jnp.float32)

    @plsc.parallel_loop(0, _G_DEG, unroll=4)
    def _group(g):
        s = pl.ds(g * L, L)
        plsc.addupdate_scatter(deg_v, [dst_v[s]], ew_v[s])
    pltpu.sync_copy(deg_v, degp_hbm.at[wid])


def _sc_degree(dst, ew):
    mesh = plsc.VectorSubcoreMesh(core_axis_name="c", subcore_axis_name="s")
    return pl.kernel(
        _sc_degree_body,
        mesh=mesh,
        compiler_params=pltpu.CompilerParams(needs_layout_passes=False),
        out_type=jax.ShapeDtypeStruct((NW, N_NODES), jnp.float32),
        scratch_types=[
            pltpu.VMEM((_E_PER_TILE,), jnp.int32),
            pltpu.VMEM((_E_PER_TILE,), jnp.float32),
            pltpu.VMEM((N_NODES,), jnp.float32),
        ],
    )(dst, ew)


def _tc_prep_body(x_ref, w1_ref, degp_ref, xwt_ref, dinv_ref):
    deg = jnp.sum(degp_ref[...], axis=0, keepdims=True) + 1.0
    dinv = jnp.where(deg > 0, lax.rsqrt(jnp.maximum(deg, 1e-12)), 0.0)
    xwt = lax.dot_general(
        w1_ref[...], x_ref[...], (((0,), (1,)), ((), ())),
        preferred_element_type=jnp.float32,
    )
    xwt_ref[...] = xwt * dinv
    dinv_ref[...] = dinv


def _tc_prep(x, w1, degp):
    return pl.pallas_call(
        _tc_prep_body,
        out_shape=[
            jax.ShapeDtypeStruct((D_HID, N_NODES), jnp.float32),
            jax.ShapeDtypeStruct((1, N_NODES), jnp.float32),
        ],
    )(x, w1, degp)


FC = 4
N_FC = D_HID // FC
N_ES = NW // N_FC
_E_PER_SLAB = N_EDGES // N_ES
_CH = 8000
_N_CH = _E_PER_SLAB // _CH
_G_MSG = _CH // L
_CHUNK_W = FC * N_NODES
_ZA = _CHUNK_W // L


def _sc_msg_body(xwt_hbm, dinv_hbm, src_hbm, dst_hbm, ew_hbm, accp_hbm,
                 xwt_v, dinv_v, acc_v, src_v, dst_v, ew_v):
    wid = lax.axis_index("s") * NC + lax.axis_index("c")
    fc = wid % N_FC
    es = wid // N_FC

    pltpu.sync_copy(xwt_hbm.at[pl.ds(fc * _CHUNK_W, _CHUNK_W)], xwt_v)
    pltpu.sync_copy(dinv_hbm, dinv_v)

    @plsc.parallel_loop(0, _ZA, unroll=8)
    def _zero(i):
        acc_v[pl.ds(i * L, L)] = jnp.zeros((L,), jnp.float32)

    ebase = es * _E_PER_SLAB

    def chunk(k, _):
        cb = ebase + k * _CH
        pltpu.sync_copy(src_hbm.at[pl.ds(cb, _CH)], src_v)
        pltpu.sync_copy(dst_hbm.at[pl.ds(cb, _CH)], dst_v)
        pltpu.sync_copy(ew_hbm.at[pl.ds(cb, _CH)], ew_v)

        @plsc.parallel_loop(0, _G_MSG, unroll=4)
        def _group(g):
            s = pl.ds(g * L, L)
            vs = src_v[s]
            vd = dst_v[s]
            scale = ew_v[s] * plsc.load_gather(dinv_v, [vd])
            for c in range(FC):
                off = jnp.int32(c * N_NODES)
                vals = plsc.load_gather(xwt_v, [vs + off])
                plsc.addupdate_scatter(acc_v, [vd + off], vals * scale)

        return 0

    lax.fori_loop(0, _N_CH, chunk, 0)
    pltpu.sync_copy(acc_v, accp_hbm.at[es, fc])


def _sc_msg(xwt_flat, dinv_flat, src, dst, ew):
    mesh = plsc.VectorSubcoreMesh(core_axis_name="c", subcore_axis_name="s")
    return pl.kernel(
        _sc_msg_body,
        mesh=mesh,
        compiler_params=pltpu.CompilerParams(needs_layout_passes=False),
        out_type=jax.ShapeDtypeStruct((N_ES, N_FC, _CHUNK_W), jnp.float32),
        scratch_types=[
            pltpu.VMEM((_CHUNK_W,), jnp.float32),
            pltpu.VMEM((N_NODES,), jnp.float32),
            pltpu.VMEM((_CHUNK_W,), jnp.float32),
            pltpu.VMEM((_CH,), jnp.int32),
            pltpu.VMEM((_CH,), jnp.int32),
            pltpu.VMEM((_CH,), jnp.float32),
        ],
    )(xwt_flat, dinv_flat, src, dst, ew)


def _tc_out_body(accp_ref, xwt_ref, dinv_ref, b1_ref, w2_ref, b2_ref, y_ref):
    h = jnp.sum(accp_ref[...], axis=0)
    h = h + dinv_ref[...] * xwt_ref[...] + b1_ref[...]
    h = jnp.maximum(h, 0.0)
    y = lax.dot_general(
        w2_ref[...], h, (((0,), (0,)), ((), ())),
        preferred_element_type=jnp.float32,
    )
    y_ref[...] = y + b2_ref[...]


def _tc_out(accp, xwt, dinv, b1, w2, b2):
    return pl.pallas_call(
        _tc_out_body,
        out_shape=jax.ShapeDtypeStruct((1, N_NODES), jnp.float32),
    )(accp, xwt, dinv, b1, w2, b2)


def kernel(x, edge_index, edge_weight, W1, b1, W2, b2):
    src = edge_index[0].astype(jnp.int32)
    dst = edge_index[1].astype(jnp.int32)
    ew = edge_weight.astype(jnp.float32)

    degp = _sc_degree(dst, ew)
    xwt, dinv = _tc_prep(x, W1, degp)
    accp = _sc_msg(xwt.reshape(-1), dinv.reshape(-1), src, dst, ew)
    accp = accp.reshape(N_ES, D_HID, N_NODES)
    y = _tc_out(accp, xwt, dinv, b1.reshape(D_HID, 1), W2, b2.reshape(1, 1))
    return y.reshape(N_NODES, 1)


# speedup vs baseline: 64.4660x; 1.2323x over previous
import functools

import jax
import jax.numpy as jnp
from jax import lax
from jax.experimental import pallas as pl
from jax.experimental.pallas import tpu as pltpu
from jax.experimental.pallas import tpu_sc as plsc

N_NODES = 10000
N_EDGES = 320000
D_FEAT = 128
D_HID = 32

NC = 2
NS = 16
NW = NC * NS
L = 16


_E_PER_TILE = N_EDGES // NW
_G_DEG = _E_PER_TILE // L
_ZN = N_NODES // L


def _sc_degree_body(dst_hbm, ew_hbm, degp_hbm, dst_v, ew_v, deg_v):
    wid = lax.axis_index("s") * NC + lax.axis_index("c")
    base = wid * _E_PER_TILE
    pltpu.sync_copy(dst_hbm.at[pl.ds(base, _E_PER_TILE)], dst_v)
    pltpu.sync_copy(ew_hbm.at[pl.ds(base, _E_PER_TILE)], ew_v)

    @plsc.parallel_loop(0, _ZN, unroll=8)
    def _zero(i):
        deg_v[pl.ds(i * L, L)] = jnp.zeros((L,), jnp.float32)

    @plsc.parallel_loop(0, _G_DEG, unroll=4)
    def _group(g):
        s = pl.ds(g * L, L)
        plsc.addupdate_scatter(deg_v, [dst_v[s]], ew_v[s])
    pltpu.sync_copy(deg_v, degp_hbm.at[wid])


def _sc_degree(dst, ew):
    mesh = plsc.VectorSubcoreMesh(core_axis_name="c", subcore_axis_name="s")
    return pl.kernel(
        _sc_degree_body,
        mesh=mesh,
        compiler_params=pltpu.CompilerParams(needs_layout_passes=False),
        out_type=jax.ShapeDtypeStruct((NW, N_NODES), jnp.float32),
        scratch_types=[
            pltpu.VMEM((_E_PER_TILE,), jnp.int32),
            pltpu.VMEM((_E_PER_TILE,), jnp.float32),
            pltpu.VMEM((N_NODES,), jnp.float32),
        ],
    )(dst, ew)


def _tc_prep_body(x_ref, w1_ref, degp_ref, xwt_ref, dinv_ref):
    deg = jnp.sum(degp_ref[...], axis=0, keepdims=True) + 1.0
    dinv = jnp.where(deg > 0, lax.rsqrt(jnp.maximum(deg, 1e-12)), 0.0)
    xwt = lax.dot_general(
        w1_ref[...], x_ref[...], (((0,), (1,)), ((), ())),
        preferred_element_type=jnp.float32,
    )
    xwt_ref[...] = xwt * dinv
    dinv_ref[...] = dinv


def _tc_prep(x, w1, degp):
    return pl.pallas_call(
        _tc_prep_body,
        out_shape=[
            jax.ShapeDtypeStruct((D_HID, N_NODES), jnp.float32),
            jax.ShapeDtypeStruct((1, N_NODES), jnp.float32),
        ],
    )(x, w1, degp)


FC = 4
N_FC = D_HID // FC
N_ES = NW // N_FC
_E_PER_SLAB = N_EDGES // N_ES
_CH = 4000
_N_CH = _E_PER_SLAB // _CH
_G_MSG = _CH // L
_CHUNK_W = FC * N_NODES
_ZA = _CHUNK_W // L


def _sc_msg_body(xwt_hbm, dinv_hbm, src_hbm, dst_hbm, ew_hbm, accp_hbm,
                 xwt_v, dinv_v, acc_v,
                 src0, src1, dst0, dst1, ew0, ew1, sem0, sem1, sem2):
    wid = lax.axis_index("s") * NC + lax.axis_index("c")
    fc = wid % N_FC
    es = wid // N_FC
    srcs, dsts, ews = (src0, src1), (dst0, dst1), (ew0, ew1)
    sems = (sem0, sem1)
    ebase = es * _E_PER_SLAB

    def start(k, b):
        cb = ebase + k * _CH
        pltpu.async_copy(src_hbm.at[pl.ds(cb, _CH)], srcs[b], sems[b])
        pltpu.async_copy(dst_hbm.at[pl.ds(cb, _CH)], dsts[b], sems[b])
        pltpu.async_copy(ew_hbm.at[pl.ds(cb, _CH)], ews[b], sems[b])

    def drain(b):
        pltpu.make_async_copy(src_hbm.at[pl.ds(0, _CH)], srcs[b], sems[b]).wait()
        pltpu.make_async_copy(dst_hbm.at[pl.ds(0, _CH)], dsts[b], sems[b]).wait()
        pltpu.make_async_copy(ew_hbm.at[pl.ds(0, _CH)], ews[b], sems[b]).wait()

    start(0, 0)
    start(1, 1)
    pltpu.async_copy(xwt_hbm.at[pl.ds(fc * _CHUNK_W, _CHUNK_W)], xwt_v, sem2)
    pltpu.async_copy(dinv_hbm, dinv_v, sem2)

    @plsc.parallel_loop(0, _ZA, unroll=8)
    def _zero(i):
        acc_v[pl.ds(i * L, L)] = jnp.zeros((L,), jnp.float32)

    pltpu.make_async_copy(xwt_hbm.at[pl.ds(0, _CHUNK_W)], xwt_v, sem2).wait()
    pltpu.make_async_copy(dinv_hbm, dinv_v, sem2).wait()

    @pl.loop(0, _N_CH, step=2)
    def _outer(k0):
        for b in range(2):
            k = k0 + b
            drain(b)
            src_v, dst_v, ew_v = srcs[b], dsts[b], ews[b]

            @plsc.parallel_loop(0, _G_MSG, unroll=4)
            def _group(g):
                s = pl.ds(g * L, L)
                vs = src_v[s]
                vd = dst_v[s]
                scale = ew_v[s] * plsc.load_gather(dinv_v, [vd])
                for c in range(FC):
                    off = jnp.int32(c * N_NODES)
                    vals = plsc.load_gather(xwt_v, [vs + off])
                    plsc.addupdate_scatter(acc_v, [vd + off], vals * scale)

            @pl.when(k + 2 < _N_CH)
            def _():
                start(k + 2, b)

    pltpu.sync_copy(acc_v, accp_hbm.at[es, fc])


def _sc_msg(xwt_flat, dinv_flat, src, dst, ew):
    mesh = plsc.VectorSubcoreMesh(core_axis_name="c", subcore_axis_name="s")
    return pl.kernel(
        _sc_msg_body,
        mesh=mesh,
        compiler_params=pltpu.CompilerParams(needs_layout_passes=False),
        out_type=jax.ShapeDtypeStruct((N_ES, N_FC, _CHUNK_W), jnp.float32),
        scratch_types=[
            pltpu.VMEM((_CHUNK_W,), jnp.float32),
            pltpu.VMEM((N_NODES,), jnp.float32),
            pltpu.VMEM((_CHUNK_W,), jnp.float32),
            pltpu.VMEM((_CH,), jnp.int32),
            pltpu.VMEM((_CH,), jnp.int32),
            pltpu.VMEM((_CH,), jnp.int32),
            pltpu.VMEM((_CH,), jnp.int32),
            pltpu.VMEM((_CH,), jnp.float32),
            pltpu.VMEM((_CH,), jnp.float32),
            pltpu.SemaphoreType.DMA,
            pltpu.SemaphoreType.DMA,
            pltpu.SemaphoreType.DMA,
        ],
    )(xwt_flat, dinv_flat, src, dst, ew)


def _tc_out_body(accp_ref, xwt_ref, dinv_ref, b1_ref, w2_ref, b2_ref, y_ref):
    h = jnp.sum(accp_ref[...], axis=0)
    h = h + dinv_ref[...] * xwt_ref[...] + b1_ref[...]
    h = jnp.maximum(h, 0.0)
    y = lax.dot_general(
        w2_ref[...], h, (((0,), (0,)), ((), ())),
        preferred_element_type=jnp.float32,
    )
    y_ref[...] = y + b2_ref[...]


def _tc_out(accp, xwt, dinv, b1, w2, b2):
    return pl.pallas_call(
        _tc_out_body,
        out_shape=jax.ShapeDtypeStruct((1, N_NODES), jnp.float32),
    )(accp, xwt, dinv, b1, w2, b2)


def kernel(x, edge_index, edge_weight, W1, b1, W2, b2):
    src = edge_index[0].astype(jnp.int32)
    dst = edge_index[1].astype(jnp.int32)
    ew = edge_weight.astype(jnp.float32)

    degp = _sc_degree(dst, ew)
    xwt, dinv = _tc_prep(x, W1, degp)
    accp = _sc_msg(xwt.reshape(-1), dinv.reshape(-1), src, dst, ew)
    accp = accp.reshape(N_ES, D_HID, N_NODES)
    y = _tc_out(accp, xwt, dinv, b1.reshape(D_HID, 1), W2, b2.reshape(1, 1))
    return y.reshape(N_NODES, 1)


# speedup vs baseline: 66.2322x; 1.0274x over previous
import functools

import jax
import jax.numpy as jnp
from jax import lax
from jax.experimental import pallas as pl
from jax.experimental.pallas import tpu as pltpu
from jax.experimental.pallas import tpu_sc as plsc

N_NODES = 10000
N_EDGES = 320000
D_FEAT = 128
D_HID = 32

NC = 2
NS = 16
NW = NC * NS
L = 16


_E_PER_TILE = N_EDGES // NW
_G_DEG = _E_PER_TILE // L
_ZN = N_NODES // L


def _sc_degree_body(dst_hbm, ew_hbm, degp_hbm, dst_v, ew_v, deg_v):
    wid = lax.axis_index("s") * NC + lax.axis_index("c")
    base = wid * _E_PER_TILE
    pltpu.sync_copy(dst_hbm.at[pl.ds(base, _E_PER_TILE)], dst_v)
    pltpu.sync_copy(ew_hbm.at[pl.ds(base, _E_PER_TILE)], ew_v)

    @plsc.parallel_loop(0, _ZN, unroll=8)
    def _zero(i):
        deg_v[pl.ds(i * L, L)] = jnp.zeros((L,), jnp.float32)

    @plsc.parallel_loop(0, _G_DEG, unroll=4)
    def _group(g):
        s = pl.ds(g * L, L)
        plsc.addupdate_scatter(deg_v, [dst_v[s]], ew_v[s])
    pltpu.sync_copy(deg_v, degp_hbm.at[wid])


def _sc_degree(dst, ew):
    mesh = plsc.VectorSubcoreMesh(core_axis_name="c", subcore_axis_name="s")
    return pl.kernel(
        _sc_degree_body,
        mesh=mesh,
        compiler_params=pltpu.CompilerParams(needs_layout_passes=False),
        out_type=jax.ShapeDtypeStruct((NW, N_NODES), jnp.float32),
        scratch_types=[
            pltpu.VMEM((_E_PER_TILE,), jnp.int32),
            pltpu.VMEM((_E_PER_TILE,), jnp.float32),
            pltpu.VMEM((N_NODES,), jnp.float32),
        ],
    )(dst, ew)


def _tc_prep_body(x_ref, w1_ref, degp_ref, src_ref, dst_ref,
                  xwt_ref, dinv_ref, eip_ref):
    deg = jnp.sum(degp_ref[...], axis=0, keepdims=True) + 1.0
    dinv = jnp.where(deg > 0, lax.rsqrt(jnp.maximum(deg, 1e-12)), 0.0)
    xwt = lax.dot_general(
        w1_ref[...], x_ref[...], (((0,), (1,)), ((), ())),
        preferred_element_type=jnp.float32,
    )
    xwt_ref[...] = xwt * dinv
    dinv_ref[...] = dinv
    eip_ref[...] = jnp.bitwise_or(lax.shift_left(src_ref[...], 14), dst_ref[...])


def _tc_prep(x, w1, degp, src2d, dst2d):
    return pl.pallas_call(
        _tc_prep_body,
        out_shape=[
            jax.ShapeDtypeStruct((D_HID, N_NODES), jnp.float32),
            jax.ShapeDtypeStruct((1, N_NODES), jnp.float32),
            jax.ShapeDtypeStruct(src2d.shape, jnp.int32),
        ],
    )(x, w1, degp, src2d, dst2d)


FC = 4
N_FC = D_HID // FC
N_ES = NW // N_FC
_E_PER_SLAB = N_EDGES // N_ES
_CH = 4000
_N_CH = _E_PER_SLAB // _CH
_G_MSG = _CH // L
_CHUNK_W = FC * N_NODES
_ZA = _CHUNK_W // L


_HALF_W = 2 * N_NODES
_G_PACK = N_NODES // L


def _sc_msg_body(xwt_hbm, dinv_hbm, eip_hbm, ew_hbm, accp_hbm,
                 stage_v, xw2_v, dinv_v, acc_v,
                 ei0, ei1, ew0, ew1, sem0, sem1, sem2):
    wid = lax.axis_index("s") * NC + lax.axis_index("c")
    fc = wid % N_FC
    es = wid // N_FC
    eis, ews = (ei0, ei1), (ew0, ew1)
    sems = (sem0, sem1)
    ebase = es * _E_PER_SLAB

    def start(k, b):
        cb = ebase + k * _CH
        pltpu.async_copy(eip_hbm.at[pl.ds(cb, _CH)], eis[b], sems[b])
        pltpu.async_copy(ew_hbm.at[pl.ds(cb, _CH)], ews[b], sems[b])

    def drain(b):
        pltpu.make_async_copy(eip_hbm.at[pl.ds(0, _CH)], eis[b], sems[b]).wait()
        pltpu.make_async_copy(ew_hbm.at[pl.ds(0, _CH)], ews[b], sems[b]).wait()

    start(0, 0)
    start(1, 1)
    pltpu.async_copy(
        xwt_hbm.at[pl.ds(fc * _CHUNK_W, _HALF_W)], stage_v, sem2)
    pltpu.async_copy(dinv_hbm, dinv_v, sem2)

    @plsc.parallel_loop(0, _ZA, unroll=8)
    def _zero(i):
        acc_v[pl.ds(i * L, L)] = jnp.zeros((L,), jnp.float32)

    pltpu.make_async_copy(xwt_hbm.at[pl.ds(0, _HALF_W)], stage_v, sem2).wait()
    pltpu.make_async_copy(dinv_hbm, dinv_v, sem2).wait()

    for h in range(2):
        hbase = h * N_NODES

        @plsc.parallel_loop(0, _G_PACK, unroll=4)
        def _pack(p):
            s = pl.ds(p * L, L)
            a = stage_v[s]
            b2 = stage_v[pl.ds(N_NODES + p * L, L)]
            pk = plsc.pack(a, b2, format=plsc.PackFormat.INTERLEAVED)
            xw2_v[pl.ds(hbase + p * L, L)] = plsc.bitcast(pk, jnp.int32)

        if h == 0:
            pltpu.async_copy(
                xwt_hbm.at[pl.ds(fc * _CHUNK_W + _HALF_W, _HALF_W)],
                stage_v, sem2)
            pltpu.make_async_copy(
                xwt_hbm.at[pl.ds(0, _HALF_W)], stage_v, sem2).wait()

    @pl.loop(0, _N_CH, step=2)
    def _outer(k0):
        for b in range(2):
            k = k0 + b
            drain(b)
            ei_v, ew_v = eis[b], ews[b]

            @plsc.parallel_loop(0, _G_MSG, unroll=4)
            def _group(g):
                s = pl.ds(g * L, L)
                ei = ei_v[s]
                vd = ei & jnp.int32(16383)
                vs = lax.shift_right_logical(ei, 14)
                scale = ew_v[s] * plsc.load_gather(dinv_v, [vd])
                for c2 in range(2):
                    pk = plsc.load_gather(xw2_v, [vs + jnp.int32(c2 * N_NODES)])
                    a, b2 = plsc.unpack(
                        plsc.bitcast(pk, jnp.bfloat16),
                        format=plsc.PackFormat.INTERLEAVED)
                    off0 = jnp.int32(2 * c2 * N_NODES)
                    off1 = jnp.int32((2 * c2 + 1) * N_NODES)
                    plsc.addupdate_scatter(acc_v, [vd + off0], a * scale)
                    plsc.addupdate_scatter(acc_v, [vd + off1], b2 * scale)

            @pl.when(k + 2 < _N_CH)
            def _():
                start(k + 2, b)

    pltpu.sync_copy(acc_v, accp_hbm.at[es, fc])


def _sc_msg(xwt_flat, dinv_flat, eip, ew):
    mesh = plsc.VectorSubcoreMesh(core_axis_name="c", subcore_axis_name="s")
    return pl.kernel(
        _sc_msg_body,
        mesh=mesh,
        compiler_params=pltpu.CompilerParams(needs_layout_passes=False),
        out_type=jax.ShapeDtypeStruct((N_ES, N_FC, _CHUNK_W), jnp.float32),
        scratch_types=[
            pltpu.VMEM((_HALF_W,), jnp.float32),
            pltpu.VMEM((_HALF_W,), jnp.int32),
            pltpu.VMEM((N_NODES,), jnp.float32),
            pltpu.VMEM((_CHUNK_W,), jnp.float32),
            pltpu.VMEM((_CH,), jnp.int32),
            pltpu.VMEM((_CH,), jnp.int32),
            pltpu.VMEM((_CH,), jnp.float32),
            pltpu.VMEM((_CH,), jnp.float32),
            pltpu.SemaphoreType.DMA,
            pltpu.SemaphoreType.DMA,
            pltpu.SemaphoreType.DMA,
        ],
    )(xwt_flat, dinv_flat, eip, ew)


def _tc_out_body(accp_ref, xwt_ref, dinv_ref, b1_ref, w2_ref, b2_ref, y_ref):
    h = jnp.sum(accp_ref[...], axis=0)
    h = h + dinv_ref[...] * xwt_ref[...] + b1_ref[...]
    h = jnp.maximum(h, 0.0)
    y = lax.dot_general(
        w2_ref[...], h, (((0,), (0,)), ((), ())),
        preferred_element_type=jnp.float32,
    )
    y_ref[...] = y + b2_ref[...]


def _tc_out(accp, xwt, dinv, b1, w2, b2):
    return pl.pallas_call(
        _tc_out_body,
        out_shape=jax.ShapeDtypeStruct((1, N_NODES), jnp.float32),
    )(accp, xwt, dinv, b1, w2, b2)


def kernel(x, edge_index, edge_weight, W1, b1, W2, b2):
    src = edge_index[0].astype(jnp.int32)
    dst = edge_index[1].astype(jnp.int32)
    ew = edge_weight.astype(jnp.float32)

    degp = _sc_degree(dst, ew)
    xwt, dinv, eip = _tc_prep(
        x, W1, degp, src.reshape(D_HID, N_NODES), dst.reshape(D_HID, N_NODES))
    accp = _sc_msg(xwt.reshape(-1), dinv.reshape(-1), eip.reshape(-1), ew)
    accp = accp.reshape(N_ES, D_HID, N_NODES)
    y = _tc_out(accp, xwt, dinv, b1.reshape(D_HID, 1), W2, b2.reshape(1, 1))
    return y.reshape(N_NODES, 1)


# speedup vs baseline: 69.2934x; 1.0462x over previous
import functools

import jax
import jax.numpy as jnp
from jax import lax
from jax.experimental import pallas as pl
from jax.experimental.pallas import tpu as pltpu
from jax.experimental.pallas import tpu_sc as plsc

N_NODES = 10000
N_EDGES = 320000
D_FEAT = 128
D_HID = 32

NC = 2
NS = 16
NW = NC * NS
L = 16


_E_PER_TILE = N_EDGES // NW
_G_DEG = _E_PER_TILE // L
_ZN = N_NODES // L


def _sc_degree_body(ei_hbm, ew_hbm, degp_hbm, eip_hbm,
                    src_v, dst_v, ew_v, deg_v, pk_v):
    wid = lax.axis_index("s") * NC + lax.axis_index("c")
    base = wid * _E_PER_TILE
    pltpu.sync_copy(ei_hbm.at[pl.ds(base, _E_PER_TILE)], src_v)
    pltpu.sync_copy(ei_hbm.at[pl.ds(N_EDGES + base, _E_PER_TILE)], dst_v)
    pltpu.sync_copy(ew_hbm.at[pl.ds(base, _E_PER_TILE)], ew_v)

    @plsc.parallel_loop(0, _ZN, unroll=8)
    def _zero(i):
        deg_v[pl.ds(i * L, L)] = jnp.zeros((L,), jnp.float32)

    @plsc.parallel_loop(0, _G_DEG, unroll=4)
    def _group(g):
        s = pl.ds(g * L, L)
        vd = dst_v[s]
        plsc.addupdate_scatter(deg_v, [vd], ew_v[s])
        pk_v[s] = jnp.bitwise_or(lax.shift_left(src_v[s], 14), vd)

    pltpu.sync_copy(deg_v, degp_hbm.at[wid])
    pltpu.sync_copy(pk_v, eip_hbm.at[pl.ds(base, _E_PER_TILE)])


def _sc_degree(ei_flat, ew):
    mesh = plsc.VectorSubcoreMesh(core_axis_name="c", subcore_axis_name="s")
    return pl.kernel(
        _sc_degree_body,
        mesh=mesh,
        compiler_params=pltpu.CompilerParams(needs_layout_passes=False),
        out_type=[
            jax.ShapeDtypeStruct((NW, N_NODES), jnp.float32),
            jax.ShapeDtypeStruct((N_EDGES,), jnp.int32),
        ],
        scratch_types=[
            pltpu.VMEM((_E_PER_TILE,), jnp.int32),
            pltpu.VMEM((_E_PER_TILE,), jnp.int32),
            pltpu.VMEM((_E_PER_TILE,), jnp.float32),
            pltpu.VMEM((N_NODES,), jnp.float32),
            pltpu.VMEM((_E_PER_TILE,), jnp.int32),
        ],
    )(ei_flat, ew)


def _tc_prep_body(x_ref, w1_ref, degp_ref, xwt_ref, dinv_ref):
    deg = jnp.sum(degp_ref[...], axis=0, keepdims=True) + 1.0
    dinv = jnp.where(deg > 0, lax.rsqrt(jnp.maximum(deg, 1e-12)), 0.0)
    xwt = lax.dot_general(
        w1_ref[...], x_ref[...], (((0,), (1,)), ((), ())),
        preferred_element_type=jnp.float32,
    )
    xwt_ref[...] = xwt * dinv
    dinv_ref[...] = dinv


def _tc_prep(x, w1, degp):
    return pl.pallas_call(
        _tc_prep_body,
        out_shape=[
            jax.ShapeDtypeStruct((D_HID, N_NODES), jnp.float32),
            jax.ShapeDtypeStruct((1, N_NODES), jnp.float32),
        ],
    )(x, w1, degp)


FC = 4
N_FC = D_HID // FC
N_ES = NW // N_FC
_E_PER_SLAB = N_EDGES // N_ES
_CH = 4000
_N_CH = _E_PER_SLAB // _CH
_G_MSG = _CH // L
_CHUNK_W = FC * N_NODES
_ZA = _CHUNK_W // L


_HALF_W = 2 * N_NODES
_G_PACK = N_NODES // L


def _sc_msg_body(xwt_hbm, dinv_hbm, eip_hbm, ew_hbm, accp_hbm,
                 stage_v, xw2_v, dinv_v, acc_v,
                 ei0, ei1, ew0, ew1, sem0, sem1, sem2):
    wid = lax.axis_index("s") * NC + lax.axis_index("c")
    fc = wid % N_FC
    es = wid // N_FC
    eis, ews = (ei0, ei1), (ew0, ew1)
    sems = (sem0, sem1)
    ebase = es * _E_PER_SLAB

    def start(k, b):
        cb = ebase + k * _CH
        pltpu.async_copy(eip_hbm.at[pl.ds(cb, _CH)], eis[b], sems[b])
        pltpu.async_copy(ew_hbm.at[pl.ds(cb, _CH)], ews[b], sems[b])

    def drain(b):
        pltpu.make_async_copy(eip_hbm.at[pl.ds(0, _CH)], eis[b], sems[b]).wait()
        pltpu.make_async_copy(ew_hbm.at[pl.ds(0, _CH)], ews[b], sems[b]).wait()

    start(0, 0)
    start(1, 1)
    pltpu.async_copy(
        xwt_hbm.at[pl.ds(fc * _CHUNK_W, _HALF_W)], stage_v, sem2)
    pltpu.async_copy(dinv_hbm, dinv_v, sem2)

    @plsc.parallel_loop(0, _ZA, unroll=8)
    def _zero(i):
        acc_v[pl.ds(i * L, L)] = jnp.zeros((L,), jnp.float32)

    pltpu.make_async_copy(xwt_hbm.at[pl.ds(0, _HALF_W)], stage_v, sem2).wait()
    pltpu.make_async_copy(dinv_hbm, dinv_v, sem2).wait()

    for h in range(2):
        hbase = h * N_NODES

        @plsc.parallel_loop(0, _G_PACK, unroll=4)
        def _pack(p):
            s = pl.ds(p * L, L)
            a = stage_v[s]
            b2 = stage_v[pl.ds(N_NODES + p * L, L)]
            pk = plsc.pack(a, b2, format=plsc.PackFormat.INTERLEAVED)
            xw2_v[pl.ds(hbase + p * L, L)] = plsc.bitcast(pk, jnp.int32)

        if h == 0:
            pltpu.async_copy(
                xwt_hbm.at[pl.ds(fc * _CHUNK_W + _HALF_W, _HALF_W)],
                stage_v, sem2)
            pltpu.make_async_copy(
                xwt_hbm.at[pl.ds(0, _HALF_W)], stage_v, sem2).wait()

    @pl.loop(0, _N_CH, step=2)
    def _outer(k0):
        for b in range(2):
            k = k0 + b
            drain(b)
            ei_v, ew_v = eis[b], ews[b]

            @plsc.parallel_loop(0, _G_MSG, unroll=4)
            def _group(g):
                s = pl.ds(g * L, L)
                ei = ei_v[s]
                vd = ei & jnp.int32(16383)
                vs = lax.shift_right_logical(ei, 14)
                scale = ew_v[s] * plsc.load_gather(dinv_v, [vd])
                for c2 in range(2):
                    pk = plsc.load_gather(xw2_v, [vs + jnp.int32(c2 * N_NODES)])
                    a, b2 = plsc.unpack(
                        plsc.bitcast(pk, jnp.bfloat16),
                        format=plsc.PackFormat.INTERLEAVED)
                    off0 = jnp.int32(2 * c2 * N_NODES)
                    off1 = jnp.int32((2 * c2 + 1) * N_NODES)
                    plsc.addupdate_scatter(acc_v, [vd + off0], a * scale)
                    plsc.addupdate_scatter(acc_v, [vd + off1], b2 * scale)

            @pl.when(k + 2 < _N_CH)
            def _():
                start(k + 2, b)

    pltpu.sync_copy(acc_v, accp_hbm.at[es, fc])


def _sc_msg(xwt_flat, dinv_flat, eip, ew):
    mesh = plsc.VectorSubcoreMesh(core_axis_name="c", subcore_axis_name="s")
    return pl.kernel(
        _sc_msg_body,
        mesh=mesh,
        compiler_params=pltpu.CompilerParams(needs_layout_passes=False),
        out_type=jax.ShapeDtypeStruct((N_ES, N_FC, _CHUNK_W), jnp.float32),
        scratch_types=[
            pltpu.VMEM((_HALF_W,), jnp.float32),
            pltpu.VMEM((_HALF_W,), jnp.int32),
            pltpu.VMEM((N_NODES,), jnp.float32),
            pltpu.VMEM((_CHUNK_W,), jnp.float32),
            pltpu.VMEM((_CH,), jnp.int32),
            pltpu.VMEM((_CH,), jnp.int32),
            pltpu.VMEM((_CH,), jnp.float32),
            pltpu.VMEM((_CH,), jnp.float32),
            pltpu.SemaphoreType.DMA,
            pltpu.SemaphoreType.DMA,
            pltpu.SemaphoreType.DMA,
        ],
    )(xwt_flat, dinv_flat, eip, ew)


def _tc_out_body(accp_ref, xwt_ref, dinv_ref, b1_ref, w2_ref, b2_ref, y_ref):
    h = jnp.sum(accp_ref[...], axis=0)
    h = h + dinv_ref[...] * xwt_ref[...] + b1_ref[...]
    h = jnp.maximum(h, 0.0)
    y = lax.dot_general(
        h, w2_ref[...], (((0,), (0,)), ((), ())),
        preferred_element_type=jnp.float32,
    )
    y_ref[...] = y + b2_ref[...]


def _tc_out(accp, xwt, dinv, b1, w2, b2):
    return pl.pallas_call(
        _tc_out_body,
        out_shape=jax.ShapeDtypeStruct((N_NODES, 1), jnp.float32),
    )(accp, xwt, dinv, b1, w2, b2)


def kernel(x, edge_index, edge_weight, W1, b1, W2, b2):
    ei = edge_index.astype(jnp.int32)
    ew = edge_weight.astype(jnp.float32)

    degp, eip = _sc_degree(ei.reshape(-1), ew)
    xwt, dinv = _tc_prep(x, W1, degp)
    accp = _sc_msg(xwt.reshape(-1), dinv.reshape(-1), eip, ew)
    accp = accp.reshape(N_ES, D_HID, N_NODES)
    return _tc_out(accp, xwt, dinv, b1.reshape(D_HID, 1), W2, b2.reshape(1, 1))


# speedup vs baseline: 69.3272x; 1.0005x over previous
import functools

import jax
import jax.numpy as jnp
from jax import lax
from jax.experimental import pallas as pl
from jax.experimental.pallas import tpu as pltpu
from jax.experimental.pallas import tpu_sc as plsc

N_NODES = 10000
N_EDGES = 320000
D_FEAT = 128
D_HID = 32

NC = 2
NS = 16
NW = NC * NS
L = 16


_E_PER_TILE = N_EDGES // NW
_G_DEG = _E_PER_TILE // L
_ZN = N_NODES // L


def _sc_degree_body(ei_hbm, ew_hbm, degp_hbm, eip_hbm,
                    src_v, dst_v, ew_v, deg_v, pk_v):
    wid = lax.axis_index("s") * NC + lax.axis_index("c")
    base = wid * _E_PER_TILE
    pltpu.sync_copy(ei_hbm.at[pl.ds(base, _E_PER_TILE)], src_v)
    pltpu.sync_copy(ei_hbm.at[pl.ds(N_EDGES + base, _E_PER_TILE)], dst_v)
    pltpu.sync_copy(ew_hbm.at[pl.ds(base, _E_PER_TILE)], ew_v)

    @plsc.parallel_loop(0, _ZN, unroll=8)
    def _zero(i):
        deg_v[pl.ds(i * L, L)] = jnp.zeros((L,), jnp.float32)

    @plsc.parallel_loop(0, _G_DEG, unroll=4)
    def _group(g):
        s = pl.ds(g * L, L)
        vd = dst_v[s]
        plsc.addupdate_scatter(deg_v, [vd], ew_v[s])
        pk_v[s] = jnp.bitwise_or(lax.shift_left(src_v[s], 14), vd)

    pltpu.sync_copy(deg_v, degp_hbm.at[wid])
    pltpu.sync_copy(pk_v, eip_hbm.at[pl.ds(base, _E_PER_TILE)])


def _sc_degree(ei_flat, ew):
    mesh = plsc.VectorSubcoreMesh(core_axis_name="c", subcore_axis_name="s")
    return pl.kernel(
        _sc_degree_body,
        mesh=mesh,
        compiler_params=pltpu.CompilerParams(needs_layout_passes=False),
        out_type=[
            jax.ShapeDtypeStruct((NW, N_NODES), jnp.float32),
            jax.ShapeDtypeStruct((N_EDGES,), jnp.int32),
        ],
        scratch_types=[
            pltpu.VMEM((_E_PER_TILE,), jnp.int32),
            pltpu.VMEM((_E_PER_TILE,), jnp.int32),
            pltpu.VMEM((_E_PER_TILE,), jnp.float32),
            pltpu.VMEM((N_NODES,), jnp.float32),
            pltpu.VMEM((_E_PER_TILE,), jnp.int32),
        ],
    )(ei_flat, ew)


def _tc_prep_body(x_ref, w1_ref, degp_ref, xwt_ref, dinv_ref):
    deg = jnp.sum(degp_ref[...], axis=0, keepdims=True) + 1.0
    dinv = jnp.where(deg > 0, lax.rsqrt(jnp.maximum(deg, 1e-12)), 0.0)
    xwt = lax.dot_general(
        w1_ref[...], x_ref[...], (((0,), (1,)), ((), ())),
        preferred_element_type=jnp.float32,
    )
    xwt_ref[...] = xwt * dinv
    dinv_ref[...] = dinv


def _tc_prep(x, w1, degp):
    return pl.pallas_call(
        _tc_prep_body,
        out_shape=[
            jax.ShapeDtypeStruct((D_HID, N_NODES), jnp.float32),
            jax.ShapeDtypeStruct((1, N_NODES), jnp.float32),
        ],
    )(x, w1, degp)


FC = 4
N_FC = D_HID // FC
N_ES = NW // N_FC
_E_PER_SLAB = N_EDGES // N_ES
_CH = 8000
_N_CH = _E_PER_SLAB // _CH
_G_MSG = _CH // L
_CHUNK_W = FC * N_NODES
_ZA = _CHUNK_W // L


_HALF_W = 2 * N_NODES
_G_PACK = N_NODES // L


def _sc_msg_body(xwt_hbm, dinv_hbm, eip_hbm, ew_hbm, accp_hbm,
                 stage_v, xw2_v, dinv_v, acc_v,
                 ei0, ei1, ew0, ew1, sem0, sem1, sem2):
    wid = lax.axis_index("s") * NC + lax.axis_index("c")
    fc = wid % N_FC
    es = wid // N_FC
    eis, ews = (ei0, ei1), (ew0, ew1)
    sems = (sem0, sem1)
    ebase = es * _E_PER_SLAB

    def start(k, b):
        cb = ebase + k * _CH
        pltpu.async_copy(eip_hbm.at[pl.ds(cb, _CH)], eis[b], sems[b])
        pltpu.async_copy(ew_hbm.at[pl.ds(cb, _CH)], ews[b], sems[b])

    def drain(b):
        pltpu.make_async_copy(eip_hbm.at[pl.ds(0, _CH)], eis[b], sems[b]).wait()
        pltpu.make_async_copy(ew_hbm.at[pl.ds(0, _CH)], ews[b], sems[b]).wait()

    start(0, 0)
    start(1, 1)
    pltpu.async_copy(
        xwt_hbm.at[pl.ds(fc * _CHUNK_W, _HALF_W)], stage_v, sem2)
    pltpu.async_copy(dinv_hbm, dinv_v, sem2)

    @plsc.parallel_loop(0, _ZA, unroll=8)
    def _zero(i):
        acc_v[pl.ds(i * L, L)] = jnp.zeros((L,), jnp.float32)

    pltpu.make_async_copy(xwt_hbm.at[pl.ds(0, _HALF_W)], stage_v, sem2).wait()
    pltpu.make_async_copy(dinv_hbm, dinv_v, sem2).wait()

    for h in range(2):
        hbase = h * N_NODES

        @plsc.parallel_loop(0, _G_PACK, unroll=4)
        def _pack(p):
            s = pl.ds(p * L, L)
            a = stage_v[s]
            b2 = stage_v[pl.ds(N_NODES + p * L, L)]
            pk = plsc.pack(a, b2, format=plsc.PackFormat.INTERLEAVED)
            xw2_v[pl.ds(hbase + p * L, L)] = plsc.bitcast(pk, jnp.int32)

        if h == 0:
            pltpu.async_copy(
                xwt_hbm.at[pl.ds(fc * _CHUNK_W + _HALF_W, _HALF_W)],
                stage_v, sem2)
            pltpu.make_async_copy(
                xwt_hbm.at[pl.ds(0, _HALF_W)], stage_v, sem2).wait()

    @pl.loop(0, _N_CH, step=2)
    def _outer(k0):
        for b in range(2):
            k = k0 + b
            drain(b)
            ei_v, ew_v = eis[b], ews[b]

            @plsc.parallel_loop(0, _G_MSG, unroll=4)
            def _group(g):
                s = pl.ds(g * L, L)
                ei = ei_v[s]
                vd = ei & jnp.int32(16383)
                vs = lax.shift_right_logical(ei, 14)
                scale = ew_v[s] * plsc.load_gather(dinv_v, [vd])
                for c2 in range(2):
                    pk = plsc.load_gather(xw2_v, [vs + jnp.int32(c2 * N_NODES)])
                    a, b2 = plsc.unpack(
                        plsc.bitcast(pk, jnp.bfloat16),
                        format=plsc.PackFormat.INTERLEAVED)
                    off0 = jnp.int32(2 * c2 * N_NODES)
                    off1 = jnp.int32((2 * c2 + 1) * N_NODES)
                    plsc.addupdate_scatter(acc_v, [vd + off0], a * scale)
                    plsc.addupdate_scatter(acc_v, [vd + off1], b2 * scale)

            @pl.when(k + 2 < _N_CH)
            def _():
                start(k + 2, b)

    pltpu.sync_copy(acc_v, accp_hbm.at[es, fc])


def _sc_msg(xwt_flat, dinv_flat, eip, ew):
    mesh = plsc.VectorSubcoreMesh(core_axis_name="c", subcore_axis_name="s")
    return pl.kernel(
        _sc_msg_body,
        mesh=mesh,
        compiler_params=pltpu.CompilerParams(needs_layout_passes=False),
        out_type=jax.ShapeDtypeStruct((N_ES, N_FC, _CHUNK_W), jnp.float32),
        scratch_types=[
            pltpu.VMEM((_HALF_W,), jnp.float32),
            pltpu.VMEM((_HALF_W,), jnp.int32),
            pltpu.VMEM((N_NODES,), jnp.float32),
            pltpu.VMEM((_CHUNK_W,), jnp.float32),
            pltpu.VMEM((_CH,), jnp.int32),
            pltpu.VMEM((_CH,), jnp.int32),
            pltpu.VMEM((_CH,), jnp.float32),
            pltpu.VMEM((_CH,), jnp.float32),
            pltpu.SemaphoreType.DMA,
            pltpu.SemaphoreType.DMA,
            pltpu.SemaphoreType.DMA,
        ],
    )(xwt_flat, dinv_flat, eip, ew)


def _tc_out_body(accp_ref, xwt_ref, dinv_ref, b1_ref, w2_ref, b2_ref, y_ref):
    h = jnp.sum(accp_ref[...], axis=0)
    h = h + dinv_ref[...] * xwt_ref[...] + b1_ref[...]
    h = jnp.maximum(h, 0.0)
    y = lax.dot_general(
        h, w2_ref[...], (((0,), (0,)), ((), ())),
        preferred_element_type=jnp.float32,
    )
    y_ref[...] = y + b2_ref[...]


def _tc_out(accp, xwt, dinv, b1, w2, b2):
    return pl.pallas_call(
        _tc_out_body,
        out_shape=jax.ShapeDtypeStruct((N_NODES, 1), jnp.float32),
    )(accp, xwt, dinv, b1, w2, b2)


def kernel(x, edge_index, edge_weight, W1, b1, W2, b2):
    ei = edge_index.astype(jnp.int32)
    ew = edge_weight.astype(jnp.float32)

    degp, eip = _sc_degree(ei.reshape(-1), ew)
    xwt, dinv = _tc_prep(x, W1, degp)
    accp = _sc_msg(xwt.reshape(-1), dinv.reshape(-1), eip, ew)
    accp = accp.reshape(N_ES, D_HID, N_NODES)
    return _tc_out(accp, xwt, dinv, b1.reshape(D_HID, 1), W2, b2.reshape(1, 1))


# speedup vs baseline: 69.3442x; 1.0002x over previous
import jax
import jax.numpy as jnp
from jax import lax
from jax.experimental import pallas as pl
from jax.experimental.pallas import tpu as pltpu
from jax.experimental.pallas import tpu_sc as plsc

N_NODES = 10000
N_EDGES = 320000
D_FEAT = 128
D_HID = 32

NC = 2
NS = 16
NW = NC * NS
L = 16


_E_PER_TILE = N_EDGES // NW
_G_DEG = _E_PER_TILE // L
_ZN = N_NODES // L


def _sc_degree_body(ei_hbm, ew_hbm, degp_hbm, eip_hbm,
                    src_v, dst_v, ew_v, deg_v, pk_v):
    wid = lax.axis_index("s") * NC + lax.axis_index("c")
    base = wid * _E_PER_TILE
    pltpu.sync_copy(ei_hbm.at[pl.ds(base, _E_PER_TILE)], src_v)
    pltpu.sync_copy(ei_hbm.at[pl.ds(N_EDGES + base, _E_PER_TILE)], dst_v)
    pltpu.sync_copy(ew_hbm.at[pl.ds(base, _E_PER_TILE)], ew_v)

    @plsc.parallel_loop(0, _ZN, unroll=8)
    def _zero(i):
        deg_v[pl.ds(i * L, L)] = jnp.zeros((L,), jnp.float32)

    @plsc.parallel_loop(0, _G_DEG, unroll=4)
    def _group(g):
        s = pl.ds(g * L, L)
        vd = dst_v[s]
        plsc.addupdate_scatter(deg_v, [vd], ew_v[s])
        pk_v[s] = jnp.bitwise_or(lax.shift_left(src_v[s], 14), vd)

    pltpu.sync_copy(deg_v, degp_hbm.at[wid])
    pltpu.sync_copy(pk_v, eip_hbm.at[pl.ds(base, _E_PER_TILE)])


def _sc_degree(ei_flat, ew):
    mesh = plsc.VectorSubcoreMesh(core_axis_name="c", subcore_axis_name="s")
    return pl.kernel(
        _sc_degree_body,
        mesh=mesh,
        compiler_params=pltpu.CompilerParams(needs_layout_passes=False),
        out_type=[
            jax.ShapeDtypeStruct((NW, N_NODES), jnp.float32),
            jax.ShapeDtypeStruct((N_EDGES,), jnp.int32),
        ],
        scratch_types=[
            pltpu.VMEM((_E_PER_TILE,), jnp.int32),
            pltpu.VMEM((_E_PER_TILE,), jnp.int32),
            pltpu.VMEM((_E_PER_TILE,), jnp.float32),
            pltpu.VMEM((N_NODES,), jnp.float32),
            pltpu.VMEM((_E_PER_TILE,), jnp.int32),
        ],
    )(ei_flat, ew)


def _tc_prep_body(x_ref, w1_ref, degp_ref, xwt_ref, dinv_ref):
    deg = jnp.sum(degp_ref[...], axis=0, keepdims=True) + 1.0
    dinv = jnp.where(deg > 0, lax.rsqrt(jnp.maximum(deg, 1e-12)), 0.0)
    xwt = lax.dot_general(
        w1_ref[...], x_ref[...], (((0,), (1,)), ((), ())),
        preferred_element_type=jnp.float32,
    )
    xwt_ref[...] = xwt * dinv
    dinv_ref[...] = dinv


def _tc_prep(x, w1, degp):
    return pl.pallas_call(
        _tc_prep_body,
        out_shape=[
            jax.ShapeDtypeStruct((D_HID, N_NODES), jnp.float32),
            jax.ShapeDtypeStruct((1, N_NODES), jnp.float32),
        ],
    )(x, w1, degp)


FC = 4
N_FC = D_HID // FC
N_ES = NW // N_FC
_E_PER_SLAB = N_EDGES // N_ES
_CH = 8000
_N_CH = _E_PER_SLAB // _CH
_G_MSG = _CH // L
_CHUNK_W = FC * N_NODES
_ZA = _CHUNK_W // L


_HALF_W = 2 * N_NODES
_G_PACK = N_NODES // L


def _sc_msg_body(xwt_hbm, dinv_hbm, eip_hbm, ew_hbm, accp_hbm,
                 stage_v, xw2_v, dinv_v, acc_v,
                 ei0, ei1, ew0, ew1, sem0, sem1, sem2):
    wid = lax.axis_index("s") * NC + lax.axis_index("c")
    fc = wid % N_FC
    es = wid // N_FC
    eis, ews = (ei0, ei1), (ew0, ew1)
    sems = (sem0, sem1)
    ebase = es * _E_PER_SLAB

    def start(k, b):
        cb = ebase + k * _CH
        pltpu.async_copy(eip_hbm.at[pl.ds(cb, _CH)], eis[b], sems[b])
        pltpu.async_copy(ew_hbm.at[pl.ds(cb, _CH)], ews[b], sems[b])

    def drain(b):
        pltpu.make_async_copy(eip_hbm.at[pl.ds(0, _CH)], eis[b], sems[b]).wait()
        pltpu.make_async_copy(ew_hbm.at[pl.ds(0, _CH)], ews[b], sems[b]).wait()

    start(0, 0)
    start(1, 1)
    pltpu.async_copy(
        xwt_hbm.at[pl.ds(fc * _CHUNK_W, _HALF_W)], stage_v, sem2)
    pltpu.async_copy(dinv_hbm, dinv_v, sem2)

    @plsc.parallel_loop(0, _ZA, unroll=8)
    def _zero(i):
        acc_v[pl.ds(i * L, L)] = jnp.zeros((L,), jnp.float32)

    pltpu.make_async_copy(xwt_hbm.at[pl.ds(0, _HALF_W)], stage_v, sem2).wait()
    pltpu.make_async_copy(dinv_hbm, dinv_v, sem2).wait()

    for h in range(2):
        hbase = h * N_NODES

        @plsc.parallel_loop(0, _G_PACK, unroll=4)
        def _pack(p):
            s = pl.ds(p * L, L)
            a = stage_v[s]
            b2 = stage_v[pl.ds(N_NODES + p * L, L)]
            pk = plsc.pack(a, b2, format=plsc.PackFormat.INTERLEAVED)
            xw2_v[pl.ds(hbase + p * L, L)] = plsc.bitcast(pk, jnp.int32)

        if h == 0:
            pltpu.async_copy(
                xwt_hbm.at[pl.ds(fc * _CHUNK_W + _HALF_W, _HALF_W)],
                stage_v, sem2)
            pltpu.make_async_copy(
                xwt_hbm.at[pl.ds(0, _HALF_W)], stage_v, sem2).wait()

    @pl.loop(0, _N_CH, step=2)
    def _outer(k0):
        for b in range(2):
            k = k0 + b
            drain(b)
            ei_v, ew_v = eis[b], ews[b]

            @plsc.parallel_loop(0, _G_MSG, unroll=4)
            def _group(g):
                s = pl.ds(g * L, L)
                ei = ei_v[s]
                vd = ei & jnp.int32(16383)
                vs = lax.shift_right_logical(ei, 14)
                scale = ew_v[s] * plsc.load_gather(dinv_v, [vd])
                for c2 in range(2):
                    pk = plsc.load_gather(xw2_v, [vs + jnp.int32(c2 * N_NODES)])
                    a, b2 = plsc.unpack(
                        plsc.bitcast(pk, jnp.bfloat16),
                        format=plsc.PackFormat.INTERLEAVED)
                    off0 = jnp.int32(2 * c2 * N_NODES)
                    off1 = jnp.int32((2 * c2 + 1) * N_NODES)
                    plsc.addupdate_scatter(acc_v, [vd + off0], a * scale)
                    plsc.addupdate_scatter(acc_v, [vd + off1], b2 * scale)

            @pl.when(k + 2 < _N_CH)
            def _():
                start(k + 2, b)

    pltpu.sync_copy(acc_v, accp_hbm.at[es, fc])


def _sc_msg(xwt_flat, dinv_flat, eip, ew):
    mesh = plsc.VectorSubcoreMesh(core_axis_name="c", subcore_axis_name="s")
    return pl.kernel(
        _sc_msg_body,
        mesh=mesh,
        compiler_params=pltpu.CompilerParams(needs_layout_passes=False),
        out_type=jax.ShapeDtypeStruct((N_ES, N_FC, _CHUNK_W), jnp.float32),
        scratch_types=[
            pltpu.VMEM((_HALF_W,), jnp.float32),
            pltpu.VMEM((_HALF_W,), jnp.int32),
            pltpu.VMEM((N_NODES,), jnp.float32),
            pltpu.VMEM((_CHUNK_W,), jnp.float32),
            pltpu.VMEM((_CH,), jnp.int32),
            pltpu.VMEM((_CH,), jnp.int32),
            pltpu.VMEM((_CH,), jnp.float32),
            pltpu.VMEM((_CH,), jnp.float32),
            pltpu.SemaphoreType.DMA,
            pltpu.SemaphoreType.DMA,
            pltpu.SemaphoreType.DMA,
        ],
    )(xwt_flat, dinv_flat, eip, ew)


def _tc_out_body(accp_ref, xwt_ref, dinv_ref, b1_ref, w2_ref, b2_ref, y_ref):
    h = jnp.sum(accp_ref[...], axis=0)
    h = h + dinv_ref[...] * xwt_ref[...] + b1_ref[...]
    h = jnp.maximum(h, 0.0)
    y = lax.dot_general(
        h, w2_ref[...], (((0,), (0,)), ((), ())),
        preferred_element_type=jnp.float32,
    )
    y_ref[...] = y + b2_ref[...]


def _tc_out(accp, xwt, dinv, b1, w2, b2):
    return pl.pallas_call(
        _tc_out_body,
        out_shape=jax.ShapeDtypeStruct((N_NODES, 1), jnp.float32),
    )(accp, xwt, dinv, b1, w2, b2)


def kernel(x, edge_index, edge_weight, W1, b1, W2, b2):
    ei = edge_index.astype(jnp.int32)
    ew = edge_weight.astype(jnp.float32)

    degp, eip = _sc_degree(ei.reshape(-1), ew)
    xwt, dinv = _tc_prep(x, W1, degp)
    accp = _sc_msg(xwt.reshape(-1), dinv.reshape(-1), eip, ew)
    accp = accp.reshape(N_ES, D_HID, N_NODES)
    return _tc_out(accp, xwt, dinv, b1.reshape(D_HID, 1), W2, b2.reshape(1, 1))
